# Initial kernel scaffold; baseline (speedup 1.0000x reference)
import jax
import jax.numpy as jnp
from jax.experimental import pallas as pl


def kernel(x, edge_index, batch, W1l, W1r, b1, W2l, W2r, b2, Wh, bh):
    raise NotImplementedError("write your pallas kernel here")


# speedup vs baseline: 4.2396x; 4.2396x over previous
import functools

import jax
import jax.numpy as jnp
from jax import lax
from jax.experimental import pallas as pl
from jax.experimental.pallas import tpu as pltpu
from jax.experimental.pallas import tpu_sc as plsc

N_NODES = 10000
N_EDGES = 320000
D = 128
D_OUT = 64
NUM_GRAPHS = 64

NC = 2
NS = 16
NW = NC * NS

CHUNK = 128
CHUNKS_PER_TILE = -(-N_EDGES // (NW * CHUNK))
E_PAD = CHUNKS_PER_TILE * NW * CHUNK
DUMMY = N_NODES
NPAD = 10112
ROWS_PER_TILE = NPAD // NS

_mesh = plsc.VectorSubcoreMesh(
    core_axis_name="c", subcore_axis_name="s", num_cores=NC, num_subcores=NS)


def _sc_body(with_cnt, *refs):
  if with_cnt:
    (x_hbm, src_hbm, dst_hbm, acc_out, cnt_out,
     src_idx, dst_idx, rows, zbuf, sem, acc_sh, cnt_v) = refs
  else:
    (x_hbm, src_hbm, dst_hbm, acc_out,
     src_idx, dst_idx, rows, zbuf, sem, acc_sh) = refs
    cnt_out = cnt_v = None

  c = lax.axis_index("c")
  s = lax.axis_index("s")
  wid = s * NC + c
  r_base = s * ROWS_PER_TILE

  zero16 = jnp.zeros((16,), jnp.float32)
  ones16 = jnp.ones((16,), jnp.float32)

  def _zrow(i, _):
    def _zlane(j, _):
      zbuf[i, pl.ds(j * 16, 16)] = zero16
      return 0
    lax.fori_loop(0, D // 16, _zlane, 0)
    return 0
  lax.fori_loop(0, 64, _zrow, 0)

  def _zacc(t, _):
    pltpu.sync_copy(zbuf, acc_sh.at[pl.ds(r_base + t * 64, 64)])
    return 0
  lax.fori_loop(0, 9, _zacc, 0)
  pltpu.sync_copy(zbuf.at[pl.ds(0, 56)], acc_sh.at[pl.ds(r_base + 576, 56)])

  if with_cnt:
    def _zcnt(i, _):
      cnt_v[pl.ds(i * 16, 16)] = zero16
      return 0
    lax.fori_loop(0, NPAD // 16, _zcnt, 0)

  plsc.subcore_barrier()

  base0 = wid * CHUNKS_PER_TILE * CHUNK

  def _edge_chunk(t, _):
    base = base0 + t * CHUNK
    pltpu.sync_copy(src_hbm.at[pl.ds(base, CHUNK)], src_idx)
    pltpu.sync_copy(dst_hbm.at[pl.ds(base, CHUNK)], dst_idx)
    pltpu.async_copy(x_hbm.at[src_idx], rows, sem).wait()
    pltpu.sync_copy(rows, acc_sh.at[dst_idx], add=True)
    if with_cnt:
      for j in range(CHUNK // 16):
        idx16 = dst_idx[pl.ds(j * 16, 16)]
        plsc.addupdate_scatter(cnt_v, [idx16], ones16)
    return 0
  lax.fori_loop(0, CHUNKS_PER_TILE, _edge_chunk, 0)

  plsc.subcore_barrier()

  def _wacc(t, _):
    r0 = r_base + t * CHUNK
    pltpu.sync_copy(acc_sh.at[pl.ds(r0, CHUNK)], rows)
    pltpu.sync_copy(rows, acc_out.at[c, pl.ds(r0, CHUNK)])
    return 0
  lax.fori_loop(0, 4, _wacc, 0)
  pltpu.sync_copy(acc_sh.at[pl.ds(r_base + 512, 120)],
                  rows.at[pl.ds(0, 120)])
  pltpu.sync_copy(rows.at[pl.ds(0, 120)],
                  acc_out.at[c, pl.ds(r_base + 512, 120)])

  if with_cnt:
    pltpu.sync_copy(cnt_v, cnt_out.at[wid])


_sc_scratch = [
    pltpu.VMEM((CHUNK,), jnp.int32),
    pltpu.VMEM((CHUNK,), jnp.int32),
    pltpu.VMEM((CHUNK, D), jnp.float32),
    pltpu.VMEM((64, D), jnp.float32),
    pltpu.SemaphoreType.DMA,
    pltpu.VMEM_SHARED((NPAD, D), jnp.float32),
]

_sc_pass1 = pl.kernel(
    functools.partial(_sc_body, True),
    out_type=(jax.ShapeDtypeStruct((NC, NPAD, D), jnp.float32),
              jax.ShapeDtypeStruct((NW, NPAD), jnp.float32)),
    mesh=_mesh,
    scratch_types=_sc_scratch + [pltpu.VMEM((NPAD,), jnp.float32)],
    compiler_params=pltpu.CompilerParams(needs_layout_passes=False),
    name="sage_edge_agg_cnt",
)

_sc_pass2 = pl.kernel(
    functools.partial(_sc_body, False),
    out_type=jax.ShapeDtypeStruct((NC, NPAD, D), jnp.float32),
    mesh=_mesh,
    scratch_types=_sc_scratch,
    compiler_params=pltpu.CompilerParams(needs_layout_passes=False),
    name="sage_edge_agg",
)


_ROWS_BLK = 400
_N_BLKS = N_NODES // _ROWS_BLK


def _combine_body(relu, p_ref, c_ref, x_ref, wl_ref, wr_ref, b_ref, o_ref):
  psum = p_ref[0] + p_ref[1]
  cnt = jnp.maximum(
      jnp.sum(c_ref[...], axis=1, keepdims=True), 1.0)
  agg = psum / cnt
  h = (jnp.dot(agg, wl_ref[...], preferred_element_type=jnp.float32)
       + jnp.dot(x_ref[...], wr_ref[...], preferred_element_type=jnp.float32)
       + b_ref[...])
  o_ref[...] = jnp.maximum(h, 0.0) if relu else h


def _tc_combine(p, cnt, x, wl, wr, b, relu):
  return pl.pallas_call(
      functools.partial(_combine_body, relu),
      grid=(_N_BLKS,),
      in_specs=[
          pl.BlockSpec((NC, _ROWS_BLK, D), lambda i: (0, i, 0)),
          pl.BlockSpec((_ROWS_BLK, NW), lambda i: (i, 0)),
          pl.BlockSpec((_ROWS_BLK, D), lambda i: (i, 0)),
          pl.BlockSpec((D, D), lambda i: (0, 0)),
          pl.BlockSpec((D, D), lambda i: (0, 0)),
          pl.BlockSpec((1, D), lambda i: (0, 0)),
      ],
      out_specs=pl.BlockSpec((_ROWS_BLK, D), lambda i: (i, 0)),
      out_shape=jax.ShapeDtypeStruct((N_NODES, D), jnp.float32),
  )(p, cnt, x, wl, wr, b)


def _pool_body(h_ref, b3_ref, wh_ref, bh_ref, o_ref, gs_ref, gc_ref):
  i = pl.program_id(0)

  @pl.when(i == 0)
  def _():
    gs_ref[...] = jnp.zeros_like(gs_ref)
    gc_ref[...] = jnp.zeros_like(gc_ref)

  bb = b3_ref[0, 0, :]
  onehot = (bb[None, :] == lax.broadcasted_iota(
      jnp.int32, (NUM_GRAPHS, _ROWS_BLK), 0)).astype(jnp.float32)
  gs_ref[...] += jnp.dot(onehot, h_ref[...],
                         preferred_element_type=jnp.float32)
  gc_ref[...] += jnp.broadcast_to(
      jnp.sum(onehot, axis=1, keepdims=True), gc_ref.shape)

  @pl.when(i == _N_BLKS - 1)
  def _():
    g = gs_ref[...] / jnp.maximum(gc_ref[...], 1.0)
    o_ref[...] = (jnp.dot(g, wh_ref[...], preferred_element_type=jnp.float32)
                  + bh_ref[...])


def _tc_pool(h, batch3, wh, bh):
  return pl.pallas_call(
      _pool_body,
      grid=(_N_BLKS,),
      in_specs=[
          pl.BlockSpec((_ROWS_BLK, D), lambda i: (i, 0)),
          pl.BlockSpec((1, 1, _ROWS_BLK), lambda i: (i, 0, 0)),
          pl.BlockSpec((D, D_OUT), lambda i: (0, 0)),
          pl.BlockSpec((1, D_OUT), lambda i: (0, 0)),
      ],
      out_specs=pl.BlockSpec((NUM_GRAPHS, D_OUT), lambda i: (0, 0)),
      out_shape=jax.ShapeDtypeStruct((NUM_GRAPHS, D_OUT), jnp.float32),
      scratch_shapes=[
          pltpu.VMEM((NUM_GRAPHS, D), jnp.float32),
          pltpu.VMEM((NUM_GRAPHS, D), jnp.float32),
      ],
  )(h, batch3, wh, bh)


@jax.jit
def kernel(x, edge_index, batch, W1l, W1r, b1, W2l, W2r, b2, Wh, bh):
  src = jnp.concatenate(
      [edge_index[0].astype(jnp.int32),
       jnp.zeros((E_PAD - N_EDGES,), jnp.int32)])
  dst = jnp.concatenate(
      [edge_index[1].astype(jnp.int32),
       jnp.full((E_PAD - N_EDGES,), DUMMY, jnp.int32)])
  batch3 = batch.astype(jnp.int32).reshape(_N_BLKS, 1, _ROWS_BLK)
  b1r = b1.reshape(1, D)
  b2r = b2.reshape(1, D)
  bhr = bh.reshape(1, D_OUT)

  p1, cnt = _sc_pass1(x, src, dst)
  cnt_t = cnt.T
  h1 = _tc_combine(p1, cnt_t, x, W1l, W1r, b1r, relu=True)
  p2 = _sc_pass2(h1, src, dst)
  h2 = _tc_combine(p2, cnt_t, h1, W2l, W2r, b2r, relu=False)
  return _tc_pool(h2, batch3, Wh, bhr)
